# baseline (speedup 1.0000x reference)
import functools

import jax
import jax.numpy as jnp
from jax import lax
from jax.experimental import pallas as pl
from jax.experimental.pallas import tpu as pltpu
from jax.experimental.pallas import tpu_sc as plsc

EMB = 64
B = 4096
T = 200
WIDE = 128
NC = 2
NS = 16
NW = NC * NS
NBC = 16
BCH = B // NBC
NTG = NW // NBC
TG = T // NTG
KS = BCH // 128
LANES = 16
EG = EMB // LANES
IP = 129
NROW = 8 * KS * 8

_mesh = plsc.VectorSubcoreMesh(core_axis_name="c", subcore_axis_name="s")


@functools.partial(
    pl.kernel,
    out_type=jax.ShapeDtypeStruct((T, 2048, 128), jnp.float32),
    mesh=_mesh,
    scratch_types=[
        pltpu.VMEM((3, KS, 128), jnp.int32),
        pltpu.VMEM((3, BCH, WIDE), jnp.float32),
        pltpu.VMEM((NROW, IP), jnp.float32),
        pltpu.SemaphoreType.DMA,
        pltpu.SemaphoreType.DMA,
        pltpu.SemaphoreType.DMA,
    ],
    compiler_params=pltpu.CompilerParams(
        use_tc_tiling_on_sc=False, needs_layout_passes=False),
)
def _gather(table_hbm, toks_hbm, out_hbm, idx_v, rows_v, tiled_v,
            isem, gsem, psem):
    wid = lax.axis_index("s") * NC + lax.axis_index("c")
    bc = lax.rem(wid, NBC)
    tg = wid // NBC
    t0 = tg * TG

    lane = lax.iota(jnp.int32, LANES)
    row_ids = [[((2 * k + lane // 8) * KS + tcl) * 8 + lax.rem(lane, 8)
                for tcl in range(KS)] for k in range(EG)]

    def idx_load(t, slot):
        tt = t0 + t
        pltpu.async_copy(
            toks_hbm.at[tt // 8, pl.ds(bc * KS, KS), lax.rem(tt, 8)],
            idx_v.at[slot], isem)

    def idx_wait():
        pltpu.make_async_copy(
            toks_hbm.at[0, pl.ds(0, KS), 0], idx_v.at[0], isem).wait()

    def start_gather(t, rb):
        slot = lax.rem(t, 3)
        for k in range(KS):
            pltpu.async_copy(
                table_hbm.at[idx_v.at[slot].at[k]],
                rows_v.at[rb].at[pl.ds(k * 128, 128)],
                gsem)

    def wait_gather():
        for k in range(KS):
            pltpu.make_async_copy(
                table_hbm.at[idx_v.at[0].at[0]],
                rows_v.at[0].at[pl.ds(0, 128)],
                gsem).wait()

    def start_put(t):
        for er in range(8):
            pltpu.async_copy(
                tiled_v.at[pl.ds(er * KS * 8, KS * 8), pl.ds(0, 128)],
                out_hbm.at[t0 + t].at[pl.ds((er * 32 + bc * KS) * 8, KS * 8)],
                psem)

    def wait_put():
        for er in range(8):
            pltpu.make_async_copy(
                tiled_v.at[pl.ds(0, KS * 8), pl.ds(0, 128)],
                out_hbm.at[0].at[pl.ds(0, KS * 8)], psem).wait()

    def transpose(rb):
        src = rows_v.at[rb]
        for k in range(EG):
            for tcl in range(KS):
                rows = row_ids[k][tcl]

                @pl.loop(0, 128, step=4)
                def _(bi0):
                    for u in range(4):
                        bi = bi0 + u
                        bv = jnp.full((LANES,), 0, jnp.int32) + bi
                        vals = src[tcl * 128 + bi, pl.ds(k * LANES, LANES)]
                        plsc.store_scatter(tiled_v, [rows, bv], vals)

    idx_load(0, 0)
    idx_wait()
    idx_load(1, 1)
    start_gather(0, 0)
    idx_wait()
    idx_load(2, 2)
    start_gather(1, 1)

    @pl.loop(0, TG)
    def _(t):
        rb = lax.rem(t, 3)
        wait_gather()

        @pl.when(t < TG - 2)
        def _():
            idx_wait()
            start_gather(t + 2, lax.rem(t + 2, 3))

            @pl.when(t < TG - 3)
            def _():
                idx_load(t + 3, lax.rem(t + 3, 3))

        @pl.when(t >= 1)
        def _():
            wait_put()

        transpose(rb)
        start_put(t)

    wait_put()


def kernel(toks, table):
    tablew = jnp.concatenate(
        [table, jnp.zeros((table.shape[0], WIDE - EMB), table.dtype)],
        axis=1)
    toks_nat = (toks.T.reshape(T // 8, 8, B // 128, 128)
                .transpose(0, 2, 1, 3))
    out = _gather(tablew, toks_nat)
    out5 = out.reshape(T, 8, 32, 8, 128)
    return (out5.transpose(2, 4, 0, 1, 3)
            .reshape(B, T, EMB))


# speedup vs baseline: 1.0019x; 1.0019x over previous
import functools

import jax
import jax.numpy as jnp
from jax import lax
from jax.experimental import pallas as pl
from jax.experimental.pallas import tpu as pltpu
from jax.experimental.pallas import tpu_sc as plsc

EMB = 64
B = 4096
T = 200
WIDE = 128
NC = 2
NS = 16
NW = NC * NS
NBC = 16
BCH = B // NBC
NTG = NW // NBC
TG = T // NTG
KS = BCH // 128
LANES = 16
EG = EMB // LANES
IP = 129
NROW = 8 * KS * 8

_mesh = plsc.VectorSubcoreMesh(core_axis_name="c", subcore_axis_name="s")


@functools.partial(
    pl.kernel,
    out_type=jax.ShapeDtypeStruct((T, 2048, 128), jnp.float32),
    mesh=_mesh,
    scratch_types=[
        pltpu.VMEM((3, KS, 128), jnp.int32),
        pltpu.VMEM((3, BCH, WIDE), jnp.float32),
        pltpu.VMEM((NROW, IP), jnp.float32),
        pltpu.SemaphoreType.DMA,
        pltpu.SemaphoreType.DMA,
        pltpu.SemaphoreType.DMA,
    ],
    compiler_params=pltpu.CompilerParams(
        use_tc_tiling_on_sc=False, needs_layout_passes=False),
)
def _gather(table_hbm, toks_hbm, out_hbm, idx_v, rows_v, tiled_v,
            isem, gsem, psem):
    wid = lax.axis_index("s") * NC + lax.axis_index("c")
    bc = lax.rem(wid, NBC)
    tg = wid // NBC
    t0 = tg * TG

    lane = lax.iota(jnp.int32, LANES)
    row_ids = [[((2 * k + lane // 8) * KS + tcl) * 8 + lax.rem(lane, 8)
                for tcl in range(KS)] for k in range(EG)]

    def idx_load(t, slot):
        tt = t0 + t
        pltpu.async_copy(
            toks_hbm.at[tt // 8, pl.ds(bc * KS, KS), lax.rem(tt, 8)],
            idx_v.at[slot], isem)

    def idx_wait():
        pltpu.make_async_copy(
            toks_hbm.at[0, pl.ds(0, KS), 0], idx_v.at[0], isem).wait()

    def start_gather(t, rb):
        slot = lax.rem(t, 3)
        for k in range(KS):
            pltpu.async_copy(
                table_hbm.at[idx_v.at[slot].at[k]],
                rows_v.at[rb].at[pl.ds(k * 128, 128)],
                gsem)

    def wait_gather():
        for k in range(KS):
            pltpu.make_async_copy(
                table_hbm.at[idx_v.at[0].at[0]],
                rows_v.at[0].at[pl.ds(0, 128)],
                gsem).wait()

    def start_put(t):
        for er in range(8):
            pltpu.async_copy(
                tiled_v.at[pl.ds(er * KS * 8, KS * 8), pl.ds(0, 128)],
                out_hbm.at[t0 + t].at[pl.ds((er * 32 + bc * KS) * 8, KS * 8)],
                psem)

    def wait_put():
        for er in range(8):
            pltpu.make_async_copy(
                tiled_v.at[pl.ds(0, KS * 8), pl.ds(0, 128)],
                out_hbm.at[0].at[pl.ds(0, KS * 8)], psem).wait()

    def transpose(rb):
        src = rows_v.at[rb]
        for k in range(EG):
            for tcl in range(KS):
                rows = row_ids[k][tcl]

                @pl.loop(0, 128, step=4)
                def _(bi0):
                    for u in range(4):
                        bi = bi0 + u
                        bv = jnp.full((LANES,), 0, jnp.int32) + bi
                        vals = src[tcl * 128 + bi, pl.ds(k * LANES, LANES)]
                        plsc.store_scatter(tiled_v, [rows, bv], vals)

    idx_load(0, 0)
    idx_wait()
    idx_load(1, 1)
    start_gather(0, 0)
    idx_wait()
    idx_load(2, 2)
    start_gather(1, 1)

    @pl.loop(0, TG)
    def _(t):
        rb = lax.rem(t, 3)
        wait_gather()

        @pl.when(t < TG - 2)
        def _():
            idx_wait()
            start_gather(t + 2, lax.rem(t + 2, 3))

            @pl.when(t < TG - 3)
            def _():
                idx_load(t + 3, lax.rem(t + 3, 3))

        @pl.when(t >= 1)
        def _():
            wait_put()

        transpose(rb)
        start_put(t)

    wait_put()


def kernel(toks, table):
    tablew = jnp.pad(table, ((0, 0), (0, WIDE - EMB)))
    toks_nat = (toks.T.reshape(T // 8, 8, B // 128, 128)
                .transpose(0, 2, 1, 3))
    out = _gather(tablew, toks_nat)
    out5 = out.reshape(T, 8, 32, 8, 128)
    return (out5.transpose(2, 4, 0, 1, 3)
            .reshape(B, T, EMB))


# speedup vs baseline: 1.0081x; 1.0061x over previous
import functools

import jax
import jax.numpy as jnp
from jax import lax
from jax.experimental import pallas as pl
from jax.experimental.pallas import tpu as pltpu
from jax.experimental.pallas import tpu_sc as plsc

EMB = 64
B = 4096
T = 200
WIDE = 128
NC = 2
NS = 16
NW = NC * NS
NBC = 16
BCH = B // NBC
NTG = NW // NBC
TG = T // NTG
KS = BCH // 128
LANES = 16
EG = EMB // LANES
IP = 129
NROW = 8 * KS * 8

_mesh = plsc.VectorSubcoreMesh(core_axis_name="c", subcore_axis_name="s")


@functools.partial(
    pl.kernel,
    out_type=jax.ShapeDtypeStruct((T, 2048, 128), jnp.float32),
    mesh=_mesh,
    scratch_types=[
        pltpu.VMEM((3, KS, 128), jnp.int32),
        pltpu.VMEM((2, BCH, WIDE), jnp.float32),
        pltpu.VMEM((NROW, IP), jnp.float32),
        pltpu.SemaphoreType.DMA,
        pltpu.SemaphoreType.DMA,
        pltpu.SemaphoreType.DMA,
    ],
    compiler_params=pltpu.CompilerParams(
        use_tc_tiling_on_sc=False, needs_layout_passes=False),
)
def _gather(table_hbm, toks_hbm, out_hbm, idx_v, rows_v, tiled_v,
            isem, gsem, psem):
    wid = lax.axis_index("s") * NC + lax.axis_index("c")
    bc = lax.rem(wid, NBC)
    tg = wid // NBC
    t0 = tg * TG

    lane = lax.iota(jnp.int32, LANES)
    row_ids = [[((2 * k + lane // 8) * KS + tcl) * 8 + lax.rem(lane, 8)
                for tcl in range(KS)] for k in range(EG)]

    def idx_load(t, slot):
        tt = t0 + t
        pltpu.async_copy(
            toks_hbm.at[tt // 8, pl.ds(bc * KS, KS), lax.rem(tt, 8)],
            idx_v.at[slot], isem)

    def idx_wait():
        pltpu.make_async_copy(
            toks_hbm.at[0, pl.ds(0, KS), 0], idx_v.at[0], isem).wait()

    def start_gather(t, rb):
        slot = lax.rem(t, 3)
        for k in range(KS):
            pltpu.async_copy(
                table_hbm.at[idx_v.at[slot].at[k]],
                rows_v.at[rb].at[pl.ds(k * 128, 128)],
                gsem)

    def wait_gather():
        for k in range(KS):
            pltpu.make_async_copy(
                table_hbm.at[idx_v.at[0].at[0]],
                rows_v.at[0].at[pl.ds(0, 128)],
                gsem).wait()

    def start_put(t):
        for er in range(8):
            pltpu.async_copy(
                tiled_v.at[pl.ds(er * KS * 8, KS * 8), pl.ds(0, 128)],
                out_hbm.at[t0 + t].at[pl.ds((er * 32 + bc * KS) * 8, KS * 8)],
                psem)

    def wait_put():
        for er in range(8):
            pltpu.make_async_copy(
                tiled_v.at[pl.ds(0, KS * 8), pl.ds(0, 128)],
                out_hbm.at[0].at[pl.ds(0, KS * 8)], psem).wait()

    def transpose(rb):
        src = rows_v.at[rb]
        for k in range(EG):
            for tcl in range(KS):
                rows = row_ids[k][tcl]

                @pl.loop(0, 128, step=4)
                def _(bi0):
                    for u in range(4):
                        bi = bi0 + u
                        bv = jnp.full((LANES,), 0, jnp.int32) + bi
                        vals = src[tcl * 128 + bi, pl.ds(k * LANES, LANES)]
                        plsc.store_scatter(tiled_v, [rows, bv], vals)

    idx_load(0, 0)
    idx_wait()
    idx_load(1, 1)
    start_gather(0, 0)

    @pl.loop(0, TG)
    def _(t):
        rb = lax.rem(t, 2)

        @pl.when(t < TG - 1)
        def _():
            idx_wait()
            start_gather(t + 1, 1 - rb)

            @pl.when(t < TG - 2)
            def _():
                idx_load(t + 2, lax.rem(t + 2, 3))

        wait_gather()

        @pl.when(t >= 1)
        def _():
            wait_put()

        transpose(rb)
        start_put(t)

    wait_put()


def kernel(toks, table):
    tablew = jnp.concatenate(
        [table, jnp.zeros((table.shape[0], WIDE - EMB), table.dtype)],
        axis=1)
    toks_nat = (toks.T.reshape(T // 8, 8, B // 128, 128)
                .transpose(0, 2, 1, 3))
    out = _gather(tablew, toks_nat)
    out5 = out.reshape(T, 8, 32, 8, 128)
    return (out5.transpose(2, 4, 0, 1, 3)
            .reshape(B, T, EMB))
